# baseline (speedup 1.0000x reference)
import functools

import jax
import jax.numpy as jnp
from jax import lax
from jax.experimental import pallas as pl
from jax.experimental.pallas import tpu as pltpu
from jax.experimental.pallas import tpu_sc as plsc

N = 10000
E = 320000
D_NODE = 128
D_EDGE = 16
H = 128
L_LAYERS = 4

NC = 2
NS = 16
NW = NC * NS
EPW = E // NW
CHUNK = 40
NCHUNK = EPW // CHUNK
SUP = 50
NSUP = NCHUNK // SUP
N_PAD = 10240
ROWS_PER_SUB = N_PAD // NS


HP = H // 2


def _bf16_bits(x):
    r = lax.bitcast_convert_type(x, jnp.uint32)
    return (r + 0x7FFF + ((r >> 16) & 1)) >> 16


def _pack_cols(t):
    lo = _bf16_bits(t[:, :HP])
    hi = _bf16_bits(t[:, HP:])
    return lax.bitcast_convert_type(lo | (hi << 16), jnp.float32)


def _node_embed_body(x_ref, w_ref, b_ref, o_ref):
    o_ref[...] = (
        jnp.dot(x_ref[...], w_ref[...], preferred_element_type=jnp.float32)
        + b_ref[...]
    )


def _node_embed(x, W_node, b_node):
    return pl.pallas_call(
        _node_embed_body,
        grid=(N // 1000,),
        in_specs=[
            pl.BlockSpec((1000, D_NODE), lambda i: (i, 0)),
            pl.BlockSpec((D_NODE, H), lambda i: (0, 0)),
            pl.BlockSpec((1, H), lambda i: (0, 0)),
        ],
        out_specs=pl.BlockSpec((1000, H), lambda i: (i, 0)),
        out_shape=jax.ShapeDtypeStruct((N, H), jnp.float32),
    )(x, W_node, b_node.reshape(1, H))


def _eproj_body(ea_ref, we_ref, be_ref, lw_ref, lb_ref, o0, o1, o2, o3):
    ea = (
        jnp.dot(ea_ref[...], we_ref[...], preferred_element_type=jnp.float32)
        + be_ref[...]
    )
    outs = (o0, o1, o2, o3)
    for l in range(L_LAYERS):
        outs[l][...] = (
            jnp.dot(ea, lw_ref[l], preferred_element_type=jnp.float32)
            + lb_ref[l, :].reshape(1, H)
        )


def _eproj(edge_attr, W_edge, b_edge, linW, linb):
    BE = 2000
    return pl.pallas_call(
        _eproj_body,
        grid=(E // BE,),
        in_specs=[
            pl.BlockSpec((BE, D_EDGE), lambda i: (i, 0)),
            pl.BlockSpec((D_EDGE, H), lambda i: (0, 0)),
            pl.BlockSpec((1, H), lambda i: (0, 0)),
            pl.BlockSpec((L_LAYERS, H, H), lambda i: (0, 0, 0)),
            pl.BlockSpec((L_LAYERS, H), lambda i: (0, 0)),
        ],
        out_specs=[pl.BlockSpec((BE, H), lambda i: (i, 0))] * L_LAYERS,
        out_shape=[jax.ShapeDtypeStruct((E, H), jnp.float32)] * L_LAYERS,
    )(edge_attr, W_edge, b_edge.reshape(1, H), linW, linb)


def _node_update_body(h_ref, a0_ref, a1_ref, w1_ref, b1_ref, w2_ref, b2_ref,
                      g_ref, bb_ref, o_ref):
    h = h_ref[...]
    z = h + a0_ref[...] + a1_ref[...]
    t = jnp.maximum(
        jnp.dot(z, w1_ref[...], preferred_element_type=jnp.float32)
        + b1_ref[...],
        0.0,
    )
    t = (
        jnp.dot(t, w2_ref[...], preferred_element_type=jnp.float32)
        + b2_ref[...]
    )
    mu = jnp.mean(t, axis=-1, keepdims=True)
    var = jnp.mean((t - mu) ** 2, axis=-1, keepdims=True)
    t = (t - mu) * lax.rsqrt(var + 1e-5) * g_ref[...] + bb_ref[...]
    o_ref[...] = h + jnp.maximum(t, 0.0)


def _node_update(h, a0, a1, W1l, b1l, W2l, b2l, gl, bl):
    row = pl.BlockSpec((1000, H), lambda i: (i, 0))
    mat = pl.BlockSpec((H, H), lambda i: (0, 0))
    vec = pl.BlockSpec((1, H), lambda i: (0, 0))
    return pl.pallas_call(
        _node_update_body,
        grid=(N // 1000,),
        in_specs=[row, row, row, mat, vec, mat, vec, vec, vec],
        out_specs=row,
        out_shape=jax.ShapeDtypeStruct((N, H), jnp.float32),
    )(h, a0, a1, W1l, b1l.reshape(1, H), W2l, b2l.reshape(1, H),
      gl.reshape(1, H), bl.reshape(1, H))


def _make_edge_pass():
    mesh = plsc.VectorSubcoreMesh(core_axis_name="c", subcore_axis_name="s")

    @functools.partial(
        pl.kernel,
        mesh=mesh,
        out_type=jax.ShapeDtypeStruct((NC * N_PAD, H), jnp.float32),
        scratch_types=[
            pltpu.VMEM((EPW,), jnp.int32),
            pltpu.VMEM((SUP, CHUNK), jnp.int32),
            pltpu.VMEM((2, CHUNK, H), jnp.float32),
            pltpu.VMEM((2, CHUNK, H), jnp.float32),
            pltpu.VMEM_SHARED((N_PAD, H), jnp.float32),
            pltpu.SemaphoreType.DMA,
            pltpu.SemaphoreType.DMA,
            pltpu.SemaphoreType.DMA,
            pltpu.SemaphoreType.DMA,
            pltpu.SemaphoreType.DMA,
            pltpu.SemaphoreType.DMA,
            pltpu.SemaphoreType.DMA,
        ],
    )
    def edge_pass(h_hbm, ep_hbm, srcr_hbm, dstr_hbm, out_hbm,
                  src_v, dst_v, epin_v, msg_v, aggr_sh,
                  gsem0, gsem1, esem0, esem1, ssem0, ssem1, isem):
        cid = lax.axis_index("c")
        sid = lax.axis_index("s")
        wid = sid * NC + cid
        gsem = (gsem0, gsem1)
        esem = (esem0, esem1)
        ssem = (ssem0, ssem1)

        cp_src = pltpu.make_async_copy(
            srcr_hbm.at[pl.ds(pl.multiple_of(wid * EPW, 8), EPW)], src_v, isem)
        cp_src.start()

        zero16 = jnp.zeros((16,), jnp.float32)

        def zrow(r, carry):
            for c in range(H // 16):
                msg_v[0, r, pl.ds(16 * c, 16)] = zero16
            return carry

        lax.fori_loop(0, CHUNK, zrow, 0)
        row0 = sid * ROWS_PER_SUB
        for j in range(ROWS_PER_SUB // CHUNK):
            pltpu.sync_copy(
                msg_v.at[0], aggr_sh.at[pl.ds(row0 + j * CHUNK, CHUNK)])
        cp_src.wait()
        plsc.subcore_barrier()

        ebase = wid * EPW

        def start_ep(c, b):
            off = pl.multiple_of(ebase + c * CHUNK, 8)
            pltpu.make_async_copy(
                ep_hbm.at[pl.ds(off, CHUNK)], epin_v.at[b], esem[b]).start()

        def wait_ep(b):
            pltpu.make_async_copy(
                ep_hbm.at[pl.ds(0, CHUNK)], epin_v.at[b], esem[b]).wait()

        def start_gadd(c, b):
            pltpu.make_async_copy(
                h_hbm.at[src_v.at[pl.ds(c * CHUNK, CHUNK)]],
                epin_v.at[b], gsem[b]).start(add=True)

        def wait_gadd(b):
            pltpu.make_async_copy(
                h_hbm.at[src_v.at[pl.ds(0, CHUNK)]],
                epin_v.at[b], gsem[b]).wait()

        def start_scatter(lc, b):
            pltpu.make_async_copy(
                msg_v.at[b], aggr_sh.at[dst_v.at[lc]], ssem[b]).start(add=True)

        def wait_scatter(b):
            pltpu.make_async_copy(
                msg_v.at[b], aggr_sh.at[dst_v.at[0]], ssem[b]).wait()

        def compute(b):
            def crow(r, inner):
                for g in range(H // 16):
                    sg = pl.ds(16 * g, 16)
                    msg_v[b, r, sg] = jnp.maximum(epin_v[b, r, sg], 0.0)
                return inner

            lax.fori_loop(0, CHUNK, crow, 0)

        def do_chunk(c, lc, b, first):
            wait_gadd(b)
            if not first:
                wait_scatter(b)
            compute(b)
            if isinstance(c, int):
                if c + 2 < NCHUNK:
                    start_ep(c + 2, b)
            else:
                @pl.when(c + 2 < NCHUNK)
                def _():
                    start_ep(c + 2, b)
            start_scatter(lc, b)
            b2 = 1 - b
            if isinstance(c, int):
                if c + 1 < NCHUNK:
                    wait_ep(b2)
                    start_gadd(c + 1, b2)
            else:
                @pl.when(c + 1 < NCHUNK)
                def _():
                    wait_ep(b2)
                    start_gadd(c + 1, b2)

        start_ep(0, 0)
        start_ep(1, 1)
        wait_ep(0)
        start_gadd(0, 0)

        for s in range(NSUP):
            base = s * SUP
            pltpu.sync_copy(dstr_hbm.at[wid, s], dst_v)
            do_chunk(base, 0, 0, True)
            do_chunk(base + 1, 1, 1, True)

            def pair(i, carry):
                c0 = base + 2 * i
                do_chunk(c0, 2 * i, 0, False)
                do_chunk(c0 + 1, 2 * i + 1, 1, False)
                return carry

            lax.fori_loop(1, SUP // 2, pair, 0)
            wait_scatter(0)
            wait_scatter(1)

        plsc.subcore_barrier()
        for j in range(ROWS_PER_SUB // CHUNK):
            rows = pl.ds(row0 + j * CHUNK, CHUNK)
            pltpu.sync_copy(aggr_sh.at[rows], msg_v.at[0])
            pltpu.sync_copy(
                msg_v.at[0],
                out_hbm.at[
                    pl.ds(pl.multiple_of(cid * N_PAD + row0 + j * CHUNK, 8),
                          CHUNK)
                ],
            )

    return edge_pass


@functools.lru_cache(maxsize=1)
def _get_edge_pass():
    return _make_edge_pass()


def kernel(x, edge_index, edge_attr, W_node, b_node, W_edge, b_edge,
           linW, linb, W1, b1, W2, b2, ln_g, ln_b):
    src = edge_index[0].astype(jnp.int32)
    dst = edge_index[1].astype(jnp.int32).reshape(NW, NSUP, SUP, CHUNK)

    h = _node_embed(x, W_node, b_node)
    ep = _eproj(edge_attr, W_edge, b_edge, linW, linb)

    edge_pass = _get_edge_pass()
    for l in range(L_LAYERS):
        aggr = edge_pass(h, ep[l], src, dst)
        h = _node_update(h, aggr[:N], aggr[N_PAD:N_PAD + N],
                         W1[l], b1[l], W2[l], b2[l],
                         ln_g[l], ln_b[l])
    return h


# speedup vs baseline: 1.2920x; 1.2920x over previous
import functools

import jax
import jax.numpy as jnp
from jax import lax
from jax.experimental import pallas as pl
from jax.experimental.pallas import tpu as pltpu
from jax.experimental.pallas import tpu_sc as plsc

N = 10000
E = 320000
D_NODE = 128
D_EDGE = 16
H = 128
L_LAYERS = 4

NC = 2
NS = 16
NW = NC * NS
EPW = E // NW
CHUNK = 40
NCHUNK = EPW // CHUNK
SUP = 50
NSUP = NCHUNK // SUP
N_PAD = 10240
ROWS_PER_SUB = N_PAD // NS


HP = H // 2


def _bf16_bits(x):
    r = lax.bitcast_convert_type(x, jnp.uint32)
    return (r + 0x7FFF + ((r >> 16) & 1)) >> 16


def _pack_cols(t):
    lo = _bf16_bits(t[:, :HP])
    hi = _bf16_bits(t[:, HP:])
    return lax.bitcast_convert_type(lo | (hi << 16), jnp.float32)


def _node_embed_body(x_ref, w_ref, b_ref, o_ref):
    o_ref[...] = (
        jnp.dot(x_ref[...], w_ref[...], preferred_element_type=jnp.float32)
        + b_ref[...]
    )


def _node_embed(x, W_node, b_node):
    return pl.pallas_call(
        _node_embed_body,
        grid=(N // 1000,),
        in_specs=[
            pl.BlockSpec((1000, D_NODE), lambda i: (i, 0)),
            pl.BlockSpec((D_NODE, H), lambda i: (0, 0)),
            pl.BlockSpec((1, H), lambda i: (0, 0)),
        ],
        out_specs=pl.BlockSpec((1000, H), lambda i: (i, 0)),
        out_shape=jax.ShapeDtypeStruct((N, H), jnp.float32),
    )(x, W_node, b_node.reshape(1, H))


def _eproj_body(ea_ref, we_ref, be_ref, lw_ref, lb_ref, o0, o1, o2, o3):
    ea = (
        jnp.dot(ea_ref[...], we_ref[...], preferred_element_type=jnp.float32)
        + be_ref[...]
    )
    outs = (o0, o1, o2, o3)
    for l in range(L_LAYERS):
        outs[l][...] = (
            jnp.dot(ea, lw_ref[l], preferred_element_type=jnp.float32)
            + lb_ref[l, :].reshape(1, H)
        )


def _eproj(edge_attr, W_edge, b_edge, linW, linb):
    BE = 2000
    return pl.pallas_call(
        _eproj_body,
        grid=(E // BE,),
        in_specs=[
            pl.BlockSpec((BE, D_EDGE), lambda i: (i, 0)),
            pl.BlockSpec((D_EDGE, H), lambda i: (0, 0)),
            pl.BlockSpec((1, H), lambda i: (0, 0)),
            pl.BlockSpec((L_LAYERS, H, H), lambda i: (0, 0, 0)),
            pl.BlockSpec((L_LAYERS, H), lambda i: (0, 0)),
        ],
        out_specs=[pl.BlockSpec((BE, H), lambda i: (i, 0))] * L_LAYERS,
        out_shape=[jax.ShapeDtypeStruct((E, H), jnp.float32)] * L_LAYERS,
    )(edge_attr, W_edge, b_edge.reshape(1, H), linW, linb)


def _node_update_body(h_ref, a0_ref, a1_ref, w1_ref, b1_ref, w2_ref, b2_ref,
                      g_ref, bb_ref, o_ref):
    h = h_ref[...]
    z = h + a0_ref[...] + a1_ref[...]
    t = jnp.maximum(
        jnp.dot(z, w1_ref[...], preferred_element_type=jnp.float32)
        + b1_ref[...],
        0.0,
    )
    t = (
        jnp.dot(t, w2_ref[...], preferred_element_type=jnp.float32)
        + b2_ref[...]
    )
    mu = jnp.mean(t, axis=-1, keepdims=True)
    var = jnp.mean((t - mu) ** 2, axis=-1, keepdims=True)
    t = (t - mu) * lax.rsqrt(var + 1e-5) * g_ref[...] + bb_ref[...]
    o_ref[...] = h + jnp.maximum(t, 0.0)


def _node_update(h, a0, a1, W1l, b1l, W2l, b2l, gl, bl):
    row = pl.BlockSpec((1000, H), lambda i: (i, 0))
    mat = pl.BlockSpec((H, H), lambda i: (0, 0))
    vec = pl.BlockSpec((1, H), lambda i: (0, 0))
    return pl.pallas_call(
        _node_update_body,
        grid=(N // 1000,),
        in_specs=[row, row, row, mat, vec, mat, vec, vec, vec],
        out_specs=row,
        out_shape=jax.ShapeDtypeStruct((N, H), jnp.float32),
    )(h, a0, a1, W1l, b1l.reshape(1, H), W2l, b2l.reshape(1, H),
      gl.reshape(1, H), bl.reshape(1, H))


def _make_edge_pass():
    mesh = plsc.VectorSubcoreMesh(core_axis_name="c", subcore_axis_name="s")

    @functools.partial(
        pl.kernel,
        mesh=mesh,
        out_type=jax.ShapeDtypeStruct((NC * N_PAD, H), jnp.float32),
        scratch_types=[
            pltpu.VMEM((EPW,), jnp.int32),
            pltpu.VMEM((SUP, CHUNK), jnp.int32),
            pltpu.VMEM((4, CHUNK, H), jnp.float32),
            pltpu.VMEM((2, CHUNK, H), jnp.float32),
            pltpu.VMEM_SHARED((N_PAD, H), jnp.float32),
            pltpu.SemaphoreType.DMA,
            pltpu.SemaphoreType.DMA,
            pltpu.SemaphoreType.DMA,
            pltpu.SemaphoreType.DMA,
            pltpu.SemaphoreType.DMA,
            pltpu.SemaphoreType.DMA,
            pltpu.SemaphoreType.DMA,
            pltpu.SemaphoreType.DMA,
            pltpu.SemaphoreType.DMA,
            pltpu.SemaphoreType.DMA,
            pltpu.SemaphoreType.DMA,
        ],
    )
    def edge_pass(h_hbm, ep_hbm, srcr_hbm, dstr_hbm, out_hbm,
                  src_v, dst_v, epin_v, msg_v, aggr_sh,
                  gsem0, gsem1, gsem2, gsem3, esem0, esem1, esem2, esem3,
                  ssem0, ssem1, isem):
        cid = lax.axis_index("c")
        sid = lax.axis_index("s")
        wid = sid * NC + cid
        gsem = (gsem0, gsem1, gsem2, gsem3)
        esem = (esem0, esem1, esem2, esem3)
        ssem = (ssem0, ssem1)

        cp_src = pltpu.make_async_copy(
            srcr_hbm.at[pl.ds(pl.multiple_of(wid * EPW, 8), EPW)], src_v, isem)
        cp_src.start()

        zero16 = jnp.zeros((16,), jnp.float32)

        def zrow(r, carry):
            for c in range(H // 16):
                msg_v[0, r, pl.ds(16 * c, 16)] = zero16
            return carry

        lax.fori_loop(0, CHUNK, zrow, 0)
        row0 = sid * ROWS_PER_SUB
        for j in range(ROWS_PER_SUB // CHUNK):
            pltpu.sync_copy(
                msg_v.at[0], aggr_sh.at[pl.ds(row0 + j * CHUNK, CHUNK)])
        cp_src.wait()
        plsc.subcore_barrier()

        ebase = wid * EPW

        def start_ep(c, b):
            off = pl.multiple_of(ebase + c * CHUNK, 8)
            pltpu.make_async_copy(
                ep_hbm.at[pl.ds(off, CHUNK)], epin_v.at[b], esem[b]).start()

        def wait_ep(b):
            pltpu.make_async_copy(
                ep_hbm.at[pl.ds(0, CHUNK)], epin_v.at[b], esem[b]).wait()

        def start_gadd(c, b):
            pltpu.make_async_copy(
                h_hbm.at[src_v.at[pl.ds(c * CHUNK, CHUNK)]],
                epin_v.at[b], gsem[b]).start(add=True)

        def wait_gadd(b):
            pltpu.make_async_copy(
                h_hbm.at[src_v.at[pl.ds(0, CHUNK)]],
                epin_v.at[b], gsem[b]).wait()

        def start_scatter(lc, b):
            pltpu.make_async_copy(
                msg_v.at[b], aggr_sh.at[dst_v.at[lc]], ssem[b]).start(add=True)

        def wait_scatter(b):
            pltpu.make_async_copy(
                msg_v.at[b], aggr_sh.at[dst_v.at[0]], ssem[b]).wait()

        def compute_relu(e, m):
            def crow(r, inner):
                for g in range(H // 16):
                    sg = pl.ds(16 * g, 16)
                    msg_v[m, r, sg] = jnp.maximum(epin_v[e, r, sg], 0.0)
                return inner

            lax.fori_loop(0, CHUNK, crow, 0)

        def do_chunk(c, lc, e, m, first):
            wait_gadd(e)
            if not first:
                wait_scatter(m)
            compute_relu(e, m)
            if isinstance(c, int):
                if c + 4 < NCHUNK:
                    start_ep(c + 4, e)
            else:
                @pl.when(c + 4 < NCHUNK)
                def _():
                    start_ep(c + 4, e)
            start_scatter(lc, m)
            e2 = (e + 2) % 4
            if isinstance(c, int):
                if c + 2 < NCHUNK:
                    wait_ep(e2)
                    start_gadd(c + 2, e2)
            else:
                @pl.when(c + 2 < NCHUNK)
                def _():
                    wait_ep(e2)
                    start_gadd(c + 2, e2)

        for e in range(4):
            start_ep(e, e)
        wait_ep(0)
        start_gadd(0, 0)
        wait_ep(1)
        start_gadd(1, 1)

        for s in range(NSUP):
            base = s * SUP
            e0 = base % 4
            pltpu.sync_copy(dstr_hbm.at[wid, s], dst_v)
            do_chunk(base, 0, e0, 0, True)
            do_chunk(base + 1, 1, (e0 + 1) % 4, 1, True)

            def quad(k, carry):
                c0 = base + 2 + 4 * k
                lc0 = 2 + 4 * k
                for j in range(4):
                    do_chunk(c0 + j, lc0 + j, (e0 + 2 + j) % 4, j % 2, False)
                return carry

            lax.fori_loop(0, (SUP - 2) // 4, quad, 0)
            wait_scatter(0)
            wait_scatter(1)

        plsc.subcore_barrier()
        for j in range(ROWS_PER_SUB // CHUNK):
            rows = pl.ds(row0 + j * CHUNK, CHUNK)
            pltpu.sync_copy(aggr_sh.at[rows], msg_v.at[0])
            pltpu.sync_copy(
                msg_v.at[0],
                out_hbm.at[
                    pl.ds(pl.multiple_of(cid * N_PAD + row0 + j * CHUNK, 8),
                          CHUNK)
                ],
            )

    return edge_pass


@functools.lru_cache(maxsize=1)
def _get_edge_pass():
    return _make_edge_pass()


def kernel(x, edge_index, edge_attr, W_node, b_node, W_edge, b_edge,
           linW, linb, W1, b1, W2, b2, ln_g, ln_b):
    src = edge_index[0].astype(jnp.int32)
    dst = edge_index[1].astype(jnp.int32).reshape(NW, NSUP, SUP, CHUNK)

    h = _node_embed(x, W_node, b_node)
    ep = _eproj(edge_attr, W_edge, b_edge, linW, linb)

    edge_pass = _get_edge_pass()
    for l in range(L_LAYERS):
        aggr = edge_pass(h, ep[l], src, dst)
        h = _node_update(h, aggr[:N], aggr[N_PAD:N_PAD + N],
                         W1[l], b1[l], W2[l], b2[l],
                         ln_g[l], ln_b[l])
    return h


# speedup vs baseline: 1.2939x; 1.0015x over previous
import functools

import jax
import jax.numpy as jnp
import numpy as np
from jax import lax
from jax.experimental import pallas as pl
from jax.experimental.pallas import tpu as pltpu
from jax.experimental.pallas import tpu_sc as plsc

N = 10000
E = 320000
D_NODE = 128
D_EDGE = 16
H = 128
L_LAYERS = 4

NC = 2
NS = 16
NW = NC * NS
EPW = E // NW
CHUNK = 40
NCHUNK = EPW // CHUNK
SUP = 50
NSUP = NCHUNK // SUP
N_PAD = 10240
ROWS_PER_SUB = N_PAD // NS


def _node_embed_body(x_ref, w_ref, b_ref, o_ref):
    o_ref[...] = (
        jnp.dot(x_ref[...], w_ref[...], preferred_element_type=jnp.float32)
        + b_ref[...]
    )


def _node_embed(x, W_node, b_node):
    return pl.pallas_call(
        _node_embed_body,
        grid=(N // 1000,),
        in_specs=[
            pl.BlockSpec((1000, D_NODE), lambda i: (i, 0)),
            pl.BlockSpec((D_NODE, H), lambda i: (0, 0)),
            pl.BlockSpec((1, H), lambda i: (0, 0)),
        ],
        out_specs=pl.BlockSpec((1000, H), lambda i: (i, 0)),
        out_shape=jax.ShapeDtypeStruct((N, H), jnp.float32),
    )(x, W_node, b_node.reshape(1, H))


def _eproj_body(ea_ref, we_ref, be_ref, lw_ref, lb_ref, o0, o1, o2, o3):
    ea = (
        jnp.dot(ea_ref[...], we_ref[...], preferred_element_type=jnp.float32)
        + be_ref[...]
    )
    outs = (o0, o1, o2, o3)
    eb = ea.astype(jnp.bfloat16)
    for l in range(L_LAYERS):
        outs[l][...] = (
            jnp.dot(eb, lw_ref[l].astype(jnp.bfloat16),
                    preferred_element_type=jnp.float32)
            + lb_ref[l, :].reshape(1, H)
        )


def _eproj(edge_attr, W_edge, b_edge, linW, linb):
    BE = 2000
    return pl.pallas_call(
        _eproj_body,
        grid=(E // BE,),
        in_specs=[
            pl.BlockSpec((BE, D_EDGE), lambda i: (i, 0)),
            pl.BlockSpec((D_EDGE, H), lambda i: (0, 0)),
            pl.BlockSpec((1, H), lambda i: (0, 0)),
            pl.BlockSpec((L_LAYERS, H, H), lambda i: (0, 0, 0)),
            pl.BlockSpec((L_LAYERS, H), lambda i: (0, 0)),
        ],
        out_specs=[pl.BlockSpec((BE, H), lambda i: (i, 0))] * L_LAYERS,
        out_shape=[jax.ShapeDtypeStruct((E, H), jnp.float32)] * L_LAYERS,
    )(edge_attr, W_edge, b_edge.reshape(1, H), linW, linb)


def _node_update_body(h_ref, a0_ref, a1_ref, w1_ref, b1_ref, w2_ref, b2_ref,
                      g_ref, bb_ref, o_ref):
    h = h_ref[...]
    z = h + a0_ref[...] + a1_ref[...]
    t = jnp.maximum(
        jnp.dot(z, w1_ref[...], preferred_element_type=jnp.float32)
        + b1_ref[...],
        0.0,
    )
    t = (
        jnp.dot(t, w2_ref[...], preferred_element_type=jnp.float32)
        + b2_ref[...]
    )
    mu = jnp.mean(t, axis=-1, keepdims=True)
    var = jnp.mean((t - mu) ** 2, axis=-1, keepdims=True)
    t = (t - mu) * lax.rsqrt(var + 1e-5) * g_ref[...] + bb_ref[...]
    o_ref[...] = h + jnp.maximum(t, 0.0)


def _node_update(h, a0, a1, W1l, b1l, W2l, b2l, gl, bl):
    row = pl.BlockSpec((1000, H), lambda i: (i, 0))
    mat = pl.BlockSpec((H, H), lambda i: (0, 0))
    vec = pl.BlockSpec((1, H), lambda i: (0, 0))
    return pl.pallas_call(
        _node_update_body,
        grid=(N // 1000,),
        in_specs=[row, row, row, mat, vec, mat, vec, vec, vec],
        out_specs=row,
        out_shape=jax.ShapeDtypeStruct((N, H), jnp.float32),
    )(h, a0, a1, W1l, b1l.reshape(1, H), W2l, b2l.reshape(1, H),
      gl.reshape(1, H), bl.reshape(1, H))


def _make_edge_pass():
    mesh = plsc.VectorSubcoreMesh(core_axis_name="c", subcore_axis_name="s")

    @functools.partial(
        pl.kernel,
        mesh=mesh,
        out_type=jax.ShapeDtypeStruct((NC * N_PAD, H), jnp.float32),
        scratch_types=[
            pltpu.VMEM((EPW,), jnp.int32),
            pltpu.VMEM((SUP, CHUNK), jnp.int32),
            pltpu.VMEM((2, CHUNK, H), jnp.float32),
            pltpu.VMEM((2, CHUNK, H), jnp.float32),
            pltpu.VMEM((2, CHUNK, H), jnp.float32),
            pltpu.VMEM_SHARED((N_PAD, H), jnp.float32),
            pltpu.SemaphoreType.DMA,
            pltpu.SemaphoreType.DMA,
            pltpu.SemaphoreType.DMA,
            pltpu.SemaphoreType.DMA,
            pltpu.SemaphoreType.DMA,
            pltpu.SemaphoreType.DMA,
            pltpu.SemaphoreType.DMA,
        ],
    )
    def edge_pass(h_hbm, ep_hbm, srcr_hbm, dstr_hbm, out_hbm,
                  src_v, dst_v, hrow_v, epin_v, msg_v, aggr_sh,
                  gsem0, gsem1, esem0, esem1, ssem0, ssem1, isem):
        cid = lax.axis_index("c")
        sid = lax.axis_index("s")
        wid = sid * NC + cid
        gsem = (gsem0, gsem1)
        esem = (esem0, esem1)
        ssem = (ssem0, ssem1)

        cp_src = pltpu.make_async_copy(
            srcr_hbm.at[pl.ds(pl.multiple_of(wid * EPW, 8), EPW)], src_v, isem)
        cp_src.start()

        zero16 = jnp.zeros((16,), jnp.float32)

        def zrow(r, carry):
            for c in range(H // 16):
                msg_v[0, r, pl.ds(16 * c, 16)] = zero16
            return carry

        lax.fori_loop(0, CHUNK, zrow, 0)
        row0 = sid * ROWS_PER_SUB
        for j in range(ROWS_PER_SUB // CHUNK):
            pltpu.sync_copy(
                msg_v.at[0], aggr_sh.at[pl.ds(row0 + j * CHUNK, CHUNK)])
        cp_src.wait()
        plsc.subcore_barrier()

        ebase = wid * EPW

        def start_fetch(c, b):
            off = pl.multiple_of(ebase + c * CHUNK, 8)
            pltpu.make_async_copy(
                ep_hbm.at[pl.ds(off, CHUNK)], epin_v.at[b], esem[b]).start()
            pltpu.make_async_copy(
                h_hbm.at[src_v.at[pl.ds(c * CHUNK, CHUNK)]],
                hrow_v.at[b], gsem[b]).start()

        def wait_fetch(b):
            pltpu.make_async_copy(
                ep_hbm.at[pl.ds(0, CHUNK)], epin_v.at[b], esem[b]).wait()
            pltpu.make_async_copy(
                h_hbm.at[src_v.at[pl.ds(0, CHUNK)]],
                hrow_v.at[b], gsem[b]).wait()

        def start_scatter(lc, b):
            pltpu.make_async_copy(
                msg_v.at[b], aggr_sh.at[dst_v.at[lc]], ssem[b]).start(add=True)

        def wait_scatter(b):
            pltpu.make_async_copy(
                msg_v.at[b], aggr_sh.at[dst_v.at[0]], ssem[b]).wait()

        def compute(b):
            def crow(r, inner):
                for g in range(H // 16):
                    s = pl.ds(16 * g, 16)
                    msg_v[b, r, s] = jnp.maximum(
                        epin_v[b, r, s] + hrow_v[b, r, s], 0.0)
                return inner

            lax.fori_loop(0, CHUNK, crow, 0)

        def do_chunk(c, lc, b, first):
            wait_fetch(b)
            if not first:
                wait_scatter(b)
            compute(b)
            if isinstance(c, int):
                if c + 2 < NCHUNK:
                    start_fetch(c + 2, b)
            else:
                @pl.when(c + 2 < NCHUNK)
                def _():
                    start_fetch(c + 2, b)
            start_scatter(lc, b)

        start_fetch(0, 0)
        start_fetch(1, 1)

        for s in range(NSUP):
            base = s * SUP
            pltpu.sync_copy(dstr_hbm.at[wid, s], dst_v)
            do_chunk(base, 0, 0, True)
            do_chunk(base + 1, 1, 1, True)

            def pair(i, carry):
                c0 = base + 2 * i
                do_chunk(c0, 2 * i, 0, False)
                do_chunk(c0 + 1, 2 * i + 1, 1, False)
                return carry

            lax.fori_loop(1, SUP // 2, pair, 0)
            wait_scatter(0)
            wait_scatter(1)

        plsc.subcore_barrier()
        for j in range(ROWS_PER_SUB // CHUNK):
            rows = pl.ds(row0 + j * CHUNK, CHUNK)
            pltpu.sync_copy(aggr_sh.at[rows], msg_v.at[0])
            pltpu.sync_copy(
                msg_v.at[0],
                out_hbm.at[
                    pl.ds(pl.multiple_of(cid * N_PAD + row0 + j * CHUNK, 8),
                          CHUNK)
                ],
            )

    return edge_pass


@functools.lru_cache(maxsize=1)
def _get_edge_pass():
    return _make_edge_pass()


def kernel(x, edge_index, edge_attr, W_node, b_node, W_edge, b_edge,
           linW, linb, W1, b1, W2, b2, ln_g, ln_b):
    src = edge_index[0].astype(jnp.int32)
    dst = edge_index[1].astype(jnp.int32).reshape(NW, NSUP, SUP, CHUNK)

    h = _node_embed(x, W_node, b_node)
    ep = _eproj(edge_attr, W_edge, b_edge, linW, linb)

    edge_pass = _get_edge_pass()
    for l in range(L_LAYERS):
        aggr = edge_pass(h, ep[l], src, dst)
        h = _node_update(h, aggr[:N], aggr[N_PAD:N_PAD + N],
                         W1[l], b1[l], W2[l], b2[l],
                         ln_g[l], ln_b[l])
    return h


# speedup vs baseline: 1.3183x; 1.0189x over previous
import functools

import jax
import jax.numpy as jnp
from jax import lax
from jax.experimental import pallas as pl
from jax.experimental.pallas import tpu as pltpu
from jax.experimental.pallas import tpu_sc as plsc

N = 10000
E = 320000
D_NODE = 128
D_EDGE = 16
H = 128
L_LAYERS = 4

NC = 2
NS = 16
NW = NC * NS
EPW = E // NW
CHUNK = 40
NCHUNK = EPW // CHUNK
SUP = 50
NSUP = NCHUNK // SUP
N_PAD = 10240
ROWS_PER_SUB = N_PAD // NS


def _node_embed_body(x_ref, w_ref, b_ref, o_ref):
    o_ref[...] = (
        jnp.dot(x_ref[...], w_ref[...], preferred_element_type=jnp.float32)
        + b_ref[...]
    )


def _node_embed(x, W_node, b_node):
    return pl.pallas_call(
        _node_embed_body,
        grid=(N // 1000,),
        in_specs=[
            pl.BlockSpec((1000, D_NODE), lambda i: (i, 0)),
            pl.BlockSpec((D_NODE, H), lambda i: (0, 0)),
            pl.BlockSpec((1, H), lambda i: (0, 0)),
        ],
        out_specs=pl.BlockSpec((1000, H), lambda i: (i, 0)),
        out_shape=jax.ShapeDtypeStruct((N, H), jnp.float32),
    )(x, W_node, b_node.reshape(1, H))


def _eproj_body(ea_ref, we_ref, be_ref, lw_ref, lb_ref, o0, o1, o2, o3):
    ea = (
        jnp.dot(ea_ref[...], we_ref[...], preferred_element_type=jnp.float32)
        + be_ref[...]
    )
    outs = (o0, o1, o2, o3)
    for l in range(L_LAYERS):
        outs[l][...] = (
            jnp.dot(ea, lw_ref[l], preferred_element_type=jnp.float32)
            + lb_ref[l, :].reshape(1, H)
        )


def _eproj(edge_attr, W_edge, b_edge, linW, linb):
    BE = 2000
    return pl.pallas_call(
        _eproj_body,
        grid=(E // BE,),
        in_specs=[
            pl.BlockSpec((BE, D_EDGE), lambda i: (i, 0)),
            pl.BlockSpec((D_EDGE, H), lambda i: (0, 0)),
            pl.BlockSpec((1, H), lambda i: (0, 0)),
            pl.BlockSpec((L_LAYERS, H, H), lambda i: (0, 0, 0)),
            pl.BlockSpec((L_LAYERS, H), lambda i: (0, 0)),
        ],
        out_specs=[pl.BlockSpec((BE, H), lambda i: (i, 0))] * L_LAYERS,
        out_shape=[jax.ShapeDtypeStruct((E, H), jnp.float32)] * L_LAYERS,
    )(edge_attr, W_edge, b_edge.reshape(1, H), linW, linb)


def _node_update_body(h_ref, a0_ref, a1_ref, w1_ref, b1_ref, w2_ref, b2_ref,
                      g_ref, bb_ref, o_ref):
    h = h_ref[...]
    z = h + a0_ref[...] + a1_ref[...]
    t = jnp.maximum(
        jnp.dot(z, w1_ref[...], preferred_element_type=jnp.float32)
        + b1_ref[...],
        0.0,
    )
    t = (
        jnp.dot(t, w2_ref[...], preferred_element_type=jnp.float32)
        + b2_ref[...]
    )
    mu = jnp.mean(t, axis=-1, keepdims=True)
    var = jnp.mean((t - mu) ** 2, axis=-1, keepdims=True)
    t = (t - mu) * lax.rsqrt(var + 1e-5) * g_ref[...] + bb_ref[...]
    o_ref[...] = h + jnp.maximum(t, 0.0)


def _node_update(h, a0, a1, W1l, b1l, W2l, b2l, gl, bl):
    row = pl.BlockSpec((1000, H), lambda i: (i, 0))
    mat = pl.BlockSpec((H, H), lambda i: (0, 0))
    vec = pl.BlockSpec((1, H), lambda i: (0, 0))
    return pl.pallas_call(
        _node_update_body,
        grid=(N // 1000,),
        in_specs=[row, row, row, mat, vec, mat, vec, vec, vec],
        out_specs=row,
        out_shape=jax.ShapeDtypeStruct((N, H), jnp.float32),
    )(h, a0, a1, W1l, b1l.reshape(1, H), W2l, b2l.reshape(1, H),
      gl.reshape(1, H), bl.reshape(1, H))


def _make_edge_pass():
    mesh = plsc.VectorSubcoreMesh(core_axis_name="c", subcore_axis_name="s")

    @functools.partial(
        pl.kernel,
        mesh=mesh,
        out_type=jax.ShapeDtypeStruct((NC * N_PAD, H), jnp.float32),
        scratch_types=[
            pltpu.VMEM((EPW,), jnp.int32),
            pltpu.VMEM((SUP, CHUNK), jnp.int32),
            pltpu.VMEM((2, CHUNK, H), jnp.float32),
            pltpu.VMEM((2, CHUNK, H), jnp.float32),
            pltpu.VMEM((2, CHUNK, H), jnp.float32),
            pltpu.VMEM_SHARED((N_PAD, H), jnp.float32),
            pltpu.SemaphoreType.DMA,
            pltpu.SemaphoreType.DMA,
            pltpu.SemaphoreType.DMA,
            pltpu.SemaphoreType.DMA,
            pltpu.SemaphoreType.DMA,
            pltpu.SemaphoreType.DMA,
            pltpu.SemaphoreType.DMA,
        ],
    )
    def edge_pass(h_hbm, ep_hbm, srcr_hbm, dstr_hbm, out_hbm,
                  src_v, dst_v, hrow_v, epin_v, msg_v, aggr_sh,
                  gsem0, gsem1, esem0, esem1, ssem0, ssem1, isem):
        cid = lax.axis_index("c")
        sid = lax.axis_index("s")
        wid = sid * NC + cid
        gsem = (gsem0, gsem1)
        esem = (esem0, esem1)
        ssem = (ssem0, ssem1)

        cp_src = pltpu.make_async_copy(
            srcr_hbm.at[pl.ds(pl.multiple_of(wid * EPW, 8), EPW)], src_v, isem)
        cp_src.start()

        zero16 = jnp.zeros((16,), jnp.float32)

        def zrow(r, carry):
            for c in range(H // 16):
                msg_v[0, r, pl.ds(16 * c, 16)] = zero16
            return carry

        lax.fori_loop(0, CHUNK, zrow, 0)
        row0 = sid * ROWS_PER_SUB
        for j in range(ROWS_PER_SUB // CHUNK):
            pltpu.sync_copy(
                msg_v.at[0], aggr_sh.at[pl.ds(row0 + j * CHUNK, CHUNK)])
        cp_src.wait()
        plsc.subcore_barrier()

        ebase = wid * EPW

        def start_fetch(c, b):
            off = pl.multiple_of(ebase + c * CHUNK, 8)
            pltpu.make_async_copy(
                ep_hbm.at[pl.ds(off, CHUNK)], epin_v.at[b], esem[b]).start()
            pltpu.make_async_copy(
                h_hbm.at[src_v.at[pl.ds(c * CHUNK, CHUNK)]],
                hrow_v.at[b], gsem[b]).start()

        def wait_fetch(b):
            pltpu.make_async_copy(
                ep_hbm.at[pl.ds(0, CHUNK)], epin_v.at[b], esem[b]).wait()
            pltpu.make_async_copy(
                h_hbm.at[src_v.at[pl.ds(0, CHUNK)]],
                hrow_v.at[b], gsem[b]).wait()

        def start_scatter(lc, b):
            pltpu.make_async_copy(
                msg_v.at[b], aggr_sh.at[dst_v.at[lc]], ssem[b]).start(add=True)

        def wait_scatter(b):
            pltpu.make_async_copy(
                msg_v.at[b], aggr_sh.at[dst_v.at[0]], ssem[b]).wait()

        def compute(b):
            def crow(r, inner):
                for g in range(H // 16):
                    s = pl.ds(16 * g, 16)
                    msg_v[b, r, s] = jnp.maximum(
                        epin_v[b, r, s] + hrow_v[b, r, s], 0.0)
                return inner

            lax.fori_loop(0, CHUNK, crow, 0)

        def do_chunk(c, lc, b, first):
            wait_fetch(b)
            if not first:
                wait_scatter(b)
            compute(b)
            if isinstance(c, int):
                if c + 2 < NCHUNK:
                    start_fetch(c + 2, b)
            else:
                @pl.when(c + 2 < NCHUNK)
                def _():
                    start_fetch(c + 2, b)
            start_scatter(lc, b)

        start_fetch(0, 0)
        start_fetch(1, 1)

        for s in range(NSUP):
            base = s * SUP
            pltpu.sync_copy(dstr_hbm.at[wid, s], dst_v)
            do_chunk(base, 0, 0, True)
            do_chunk(base + 1, 1, 1, True)

            def pair(i, carry):
                c0 = base + 2 * i
                do_chunk(c0, 2 * i, 0, False)
                do_chunk(c0 + 1, 2 * i + 1, 1, False)
                return carry

            lax.fori_loop(1, SUP // 2, pair, 0)
            wait_scatter(0)
            wait_scatter(1)

        plsc.subcore_barrier()
        for j in range(ROWS_PER_SUB // CHUNK):
            rows = pl.ds(row0 + j * CHUNK, CHUNK)
            pltpu.sync_copy(aggr_sh.at[rows], msg_v.at[0])
            pltpu.sync_copy(
                msg_v.at[0],
                out_hbm.at[
                    pl.ds(pl.multiple_of(cid * N_PAD + row0 + j * CHUNK, 8),
                          CHUNK)
                ],
            )

    return edge_pass


@functools.lru_cache(maxsize=1)
def _get_edge_pass():
    return _make_edge_pass()


def kernel(x, edge_index, edge_attr, W_node, b_node, W_edge, b_edge,
           linW, linb, W1, b1, W2, b2, ln_g, ln_b):
    src = edge_index[0].astype(jnp.int32)
    dst = edge_index[1].astype(jnp.int32).reshape(NW, NSUP, SUP, CHUNK)

    h = _node_embed(x, W_node, b_node)
    ep = _eproj(edge_attr, W_edge, b_edge, linW, linb)

    edge_pass = _get_edge_pass()
    for l in range(L_LAYERS):
        aggr = edge_pass(h, ep[l], src, dst)
        h = _node_update(h, aggr[:N], aggr[N_PAD:N_PAD + N],
                         W1[l], b1[l], W2[l], b2[l],
                         ln_g[l], ln_b[l])
    return h
